# baseline (device time: 432087 ns/iter reference)
import jax
import jax.numpy as jnp
from jax import lax
from jax.experimental import pallas as pl
from jax.experimental.pallas import tpu as pltpu

N_DEV = 4
CHUNK = 512
N_CHUNK = 2048 // CHUNK


def _a2a_bf16(x):
    m_total, k_shard = x.shape
    m_per = m_total // N_DEV

    def body(x_ref, recv_ref, ld, snd, ld_sems, op_sems, recv_sems):
        my = lax.axis_index("i")

        barrier = pltpu.get_barrier_semaphore()
        for d in range(1, N_DEV):
            pl.semaphore_signal(
                barrier, inc=1,
                device_id=((my + d) % N_DEV,),
                device_id_type=pl.DeviceIdType.MESH,
            )
        pl.semaphore_wait(barrier, N_DEV - 1)

        total = N_DEV * N_CHUNK
        d_order = (1, 2, 3, 0)

        def load(c, slot):
            di, ci = divmod(c, N_CHUNK)
            d = d_order[di]
            blk = (my + d) % N_DEV
            return pltpu.make_async_copy(
                x_ref.at[pl.ds(blk * m_per + ci * CHUNK, CHUNK), :],
                ld.at[slot],
                ld_sems.at[slot],
            )

        ops = []

        def issue(c, slot):
            di, ci = divmod(c, N_CHUNK)
            d = d_order[di]
            if d == 0:
                op = pltpu.make_async_copy(
                    snd.at[slot],
                    recv_ref.at[my, pl.ds(ci * CHUNK, CHUNK), :],
                    op_sems.at[slot],
                )
                op.start()
                ops.append(("copy", op))
            else:
                tgt = (my + d) % N_DEV
                op = pltpu.make_async_remote_copy(
                    src_ref=snd.at[slot],
                    dst_ref=recv_ref.at[my, pl.ds(ci * CHUNK, CHUNK), :],
                    send_sem=op_sems.at[slot],
                    recv_sem=recv_sems.at[my, ci],
                    device_id=(tgt,),
                    device_id_type=pl.DeviceIdType.MESH,
                )
                op.start()
                ops.append(("rdma", op))

        load(0, 0).start()
        for c in range(total):
            slot = c % 2
            if c + 1 < total:
                load(c + 1, (c + 1) % 2).start()
            load(c, slot).wait()
            if c >= 2:
                kind, op = ops[c - 2]
                if kind == "copy":
                    op.wait()
                else:
                    op.wait_send()
            snd[slot] = ld[slot].astype(jnp.bfloat16)
            issue(c, slot)
        for kind, op in ops[-2:]:
            if kind == "copy":
                op.wait()
            else:
                op.wait_send()

        for d in range(1, N_DEV):
            src = (my + d) % N_DEV
            for ci in range(N_CHUNK):
                recv = pltpu.make_async_remote_copy(
                    src_ref=snd.at[0],
                    dst_ref=recv_ref.at[src, pl.ds(ci * CHUNK, CHUNK), :],
                    send_sem=op_sems.at[0],
                    recv_sem=recv_sems.at[src, ci],
                    device_id=(src,),
                    device_id_type=pl.DeviceIdType.MESH,
                )
                recv.wait_recv()

    return pl.pallas_call(
        body,
        out_shape=jax.ShapeDtypeStruct((N_DEV, m_per, k_shard), jnp.bfloat16),
        in_specs=[pl.BlockSpec(memory_space=pl.ANY)],
        out_specs=pl.BlockSpec(memory_space=pl.ANY),
        scratch_shapes=[
            pltpu.VMEM((2, CHUNK, k_shard), jnp.float32),
            pltpu.VMEM((2, CHUNK, k_shard), jnp.bfloat16),
            pltpu.SemaphoreType.DMA((2,)),
            pltpu.SemaphoreType.DMA((2,)),
            pltpu.SemaphoreType.DMA((N_DEV, N_CHUNK)),
        ],
        compiler_params=pltpu.CompilerParams(
            collective_id=0,
            vmem_limit_bytes=60 * 1024 * 1024,
        ),
    )(x)


def _gemm(recv, w):
    _, m_per, k_shard = recv.shape
    k_total, n_total = w.shape
    bk, bn = 512, 2048
    n_grid = n_total // bn
    k_grid = k_total // bk
    k_per = k_shard // bk

    def body(x_ref, w_ref, o_ref):
        k = pl.program_id(1)

        @pl.when(k == 0)
        def _():
            o_ref[...] = jnp.zeros_like(o_ref)

        o_ref[...] += jnp.dot(
            x_ref[0], w_ref[...].astype(jnp.bfloat16),
            preferred_element_type=jnp.float32,
        )

    return pl.pallas_call(
        body,
        grid=(n_grid, k_grid),
        in_specs=[
            pl.BlockSpec((1, m_per, bk), lambda n, k: (k // k_per, 0, k % k_per)),
            pl.BlockSpec((bk, bn), lambda n, k: (k, n)),
        ],
        out_specs=pl.BlockSpec((m_per, bn), lambda n, k: (0, n)),
        out_shape=jax.ShapeDtypeStruct((m_per, n_total), jnp.float32),
        compiler_params=pltpu.CompilerParams(
            dimension_semantics=("arbitrary", "arbitrary"),
            vmem_limit_bytes=60 * 1024 * 1024,
        ),
    )(recv, w)


def kernel(x, w_mat):
    recv = _a2a_bf16(x)
    return _gemm(recv, w_mat)


# device time: 425533 ns/iter; 1.0154x vs baseline; 1.0154x over previous
import jax
import jax.numpy as jnp
from jax import lax
from jax.experimental import pallas as pl
from jax.experimental.pallas import tpu as pltpu

N_DEV = 4
CHUNK = 512
N_CHUNK = 2048 // CHUNK


def _a2a_bf16(x):
    m_total, k_shard = x.shape
    m_per = m_total // N_DEV

    def body(x_ref, recv_ref, ld, snd, ld_sems, op_sems, recv_sems):
        my = lax.axis_index("i")

        barrier = pltpu.get_barrier_semaphore()
        for d in range(1, N_DEV):
            pl.semaphore_signal(
                barrier, inc=1,
                device_id=((my + d) % N_DEV,),
                device_id_type=pl.DeviceIdType.MESH,
            )
        pl.semaphore_wait(barrier, N_DEV - 1)

        total = N_DEV * N_CHUNK
        d_order = (1, 2, 3, 0)

        def load(c, slot):
            di, ci = divmod(c, N_CHUNK)
            d = d_order[di]
            blk = (my + d) % N_DEV
            return pltpu.make_async_copy(
                x_ref.at[pl.ds(blk * m_per + ci * CHUNK, CHUNK), :],
                ld.at[slot],
                ld_sems.at[slot],
            )

        ops = []

        def issue(c, slot):
            di, ci = divmod(c, N_CHUNK)
            d = d_order[di]
            if d == 0:
                op = pltpu.make_async_copy(
                    snd.at[slot],
                    recv_ref.at[my, pl.ds(ci * CHUNK, CHUNK), :],
                    op_sems.at[slot],
                )
                op.start()
                ops.append(("copy", op))
            else:
                tgt = (my + d) % N_DEV
                op = pltpu.make_async_remote_copy(
                    src_ref=snd.at[slot],
                    dst_ref=recv_ref.at[my, pl.ds(ci * CHUNK, CHUNK), :],
                    send_sem=op_sems.at[slot],
                    recv_sem=recv_sems.at[my, ci],
                    device_id=(tgt,),
                    device_id_type=pl.DeviceIdType.MESH,
                )
                op.start()
                ops.append(("rdma", op))

        load(0, 0).start()
        for c in range(total):
            slot = c % 2
            if c + 1 < total:
                load(c + 1, (c + 1) % 2).start()
            load(c, slot).wait()
            if c >= 2:
                kind, op = ops[c - 2]
                if kind == "copy":
                    op.wait()
                else:
                    op.wait_send()
            snd[slot] = ld[slot].astype(jnp.bfloat16)
            issue(c, slot)
        for kind, op in ops[-2:]:
            if kind == "copy":
                op.wait()
            else:
                op.wait_send()

        for d in range(1, N_DEV):
            src = (my + d) % N_DEV
            for ci in range(N_CHUNK):
                recv = pltpu.make_async_remote_copy(
                    src_ref=snd.at[0],
                    dst_ref=recv_ref.at[src, pl.ds(ci * CHUNK, CHUNK), :],
                    send_sem=op_sems.at[0],
                    recv_sem=recv_sems.at[src, ci],
                    device_id=(src,),
                    device_id_type=pl.DeviceIdType.MESH,
                )
                recv.wait_recv()

    return pl.pallas_call(
        body,
        out_shape=jax.ShapeDtypeStruct((N_DEV, m_per, k_shard), jnp.bfloat16),
        in_specs=[pl.BlockSpec(memory_space=pl.ANY)],
        out_specs=pl.BlockSpec(memory_space=pl.ANY),
        scratch_shapes=[
            pltpu.VMEM((2, CHUNK, k_shard), jnp.float32),
            pltpu.VMEM((2, CHUNK, k_shard), jnp.bfloat16),
            pltpu.SemaphoreType.DMA((2,)),
            pltpu.SemaphoreType.DMA((2,)),
            pltpu.SemaphoreType.DMA((N_DEV, N_CHUNK)),
        ],
        compiler_params=pltpu.CompilerParams(
            collective_id=0,
            vmem_limit_bytes=60 * 1024 * 1024,
        ),
    )(x)


def _gemm(recv, w):
    _, m_per, k_shard = recv.shape
    k_total, n_total = w.shape
    bk = 512
    k_grid = k_total // bk
    k_per = k_shard // bk

    def body(x_ref, w_ref, o_ref):
        k = pl.program_id(0)

        @pl.when(k == 0)
        def _():
            o_ref[...] = jnp.zeros_like(o_ref)

        o_ref[...] += jnp.dot(
            x_ref[0], w_ref[...].astype(jnp.bfloat16),
            preferred_element_type=jnp.float32,
        )

    return pl.pallas_call(
        body,
        grid=(k_grid,),
        in_specs=[
            pl.BlockSpec((1, m_per, bk), lambda k: (k // k_per, 0, k % k_per)),
            pl.BlockSpec((bk, n_total), lambda k: (k, 0)),
        ],
        out_specs=pl.BlockSpec((m_per, n_total), lambda k: (0, 0)),
        out_shape=jax.ShapeDtypeStruct((m_per, n_total), jnp.float32),
        compiler_params=pltpu.CompilerParams(
            dimension_semantics=("arbitrary",),
            vmem_limit_bytes=60 * 1024 * 1024,
        ),
    )(recv, w)


def kernel(x, w_mat):
    recv = _a2a_bf16(x)
    return _gemm(recv, w_mat)
